# Initial kernel scaffold; baseline (speedup 1.0000x reference)
import jax
import jax.numpy as jnp
from jax.experimental import pallas as pl


def kernel(edge_embedding, nbr_idx):
    raise NotImplementedError("write your pallas kernel here")


# speedup vs baseline: 2.2247x; 2.2247x over previous
import functools

import jax
import jax.numpy as jnp
from jax import lax
from jax.experimental import pallas as pl
from jax.experimental.pallas import tpu as pltpu
from jax.experimental.pallas import tpu_sc as plsc


def _build_gather(R, D, NW, NC, per_w, C):
    n_chunks = per_w // C
    mesh = plsc.VectorSubcoreMesh(core_axis_name="c", subcore_axis_name="s")

    @functools.partial(
        pl.kernel,
        mesh=mesh,
        out_type=jax.ShapeDtypeStruct((R, D), jnp.float32),
        scratch_types=[
            pltpu.VMEM((n_chunks, C), jnp.int32),
            pltpu.VMEM((C, D), jnp.float32),
            pltpu.SemaphoreType.DMA,
        ],
    )
    def k(table_hbm, idx_hbm, out_hbm, idx_v, rows_v, sem):
        wid = lax.axis_index("s") * NC + lax.axis_index("c")
        base = wid * per_w
        pltpu.sync_copy(idx_hbm.at[wid], idx_v)

        @pl.loop(0, n_chunks)
        def _body(j):
            pltpu.async_copy(table_hbm.at[idx_v.at[j]], rows_v, sem).wait()
            pltpu.sync_copy(rows_v, out_hbm.at[pl.ds(base + j * C, C)])

    return k


def kernel(edge_embedding, nbr_idx):
    B, At, Nbr, F = edge_embedding.shape
    D = Nbr * F
    R = B * At * Nbr

    info = plsc.get_sparse_core_info()
    NC, NS = info.num_cores, info.num_subcores
    NW = NC * NS
    per_w = R // NW
    C = 80

    table = edge_embedding.reshape(B * At, D)
    idx = nbr_idx.astype(jnp.int32).reshape(B, At * Nbr)
    idx = idx + (jnp.arange(B, dtype=jnp.int32) * At)[:, None]
    idx = idx.reshape(NW, per_w // C, C)

    out = _build_gather(R, D, NW, NC, per_w, C)(table, idx)
    return out.reshape(B, At, Nbr, Nbr, F)


# speedup vs baseline: 2.2535x; 1.0129x over previous
import functools

import jax
import jax.numpy as jnp
from jax import lax
from jax.experimental import pallas as pl
from jax.experimental.pallas import tpu as pltpu
from jax.experimental.pallas import tpu_sc as plsc

_C = 80


def _build_gather(R, D, NC, per_w):
    n = per_w // _C
    mesh = plsc.VectorSubcoreMesh(core_axis_name="c", subcore_axis_name="s")

    @functools.partial(
        pl.kernel,
        mesh=mesh,
        out_type=jax.ShapeDtypeStruct((R, D), jnp.float32),
        scratch_types=[
            pltpu.VMEM((n, _C), jnp.int32),
            [pltpu.VMEM((_C, D), jnp.float32) for _ in range(2)],
            [pltpu.SemaphoreType.DMA for _ in range(2)],
            [pltpu.SemaphoreType.DMA for _ in range(2)],
        ],
    )
    def k(table_hbm, idx_hbm, out_hbm, idx_v, bufs, gsems, ssems):
        wid = lax.axis_index("s") * NC + lax.axis_index("c")
        base = wid * per_w
        pltpu.sync_copy(idx_hbm.at[wid], idx_v)

        def gather_start(j, b):
            pltpu.async_copy(table_hbm.at[idx_v.at[j]], bufs[b], gsems[b])

        def gather_wait(j, b):
            pltpu.make_async_copy(
                table_hbm.at[idx_v.at[j]], bufs[b], gsems[b]).wait()

        def scatter_start(j, b):
            pltpu.async_copy(
                bufs[b], out_hbm.at[pl.ds(base + j * _C, _C)], ssems[b])

        def scatter_wait(j, b):
            pltpu.make_async_copy(
                bufs[b], out_hbm.at[pl.ds(base + j * _C, _C)], ssems[b]).wait()

        def visit(j, b, first=False, fire_next=True):
            gather_wait(j, b)
            scatter_start(j, b)
            if not first:
                scatter_wait(j - 1, 1 - b)
            if fire_next:
                gather_start(j + 1, 1 - b)

        gather_start(0, 0)
        visit(0, 0, first=True)

        @pl.loop(0, (n - 3) // 2)
        def _body(i):
            j = 1 + 2 * i
            visit(j, 1)
            visit(j + 1, 0)

        visit(n - 2, 1)
        visit(n - 1, 0, fire_next=False)
        scatter_wait(n - 1, 0)

    return k


def kernel(edge_embedding, nbr_idx):
    B, At, Nbr, F = edge_embedding.shape
    D = Nbr * F
    R = B * At * Nbr

    info = plsc.get_sparse_core_info()
    NC, NS = info.num_cores, info.num_subcores
    NW = NC * NS
    per_w = R // NW

    table = edge_embedding.reshape(B * At, D)
    idx = nbr_idx.astype(jnp.int32).reshape(B, At * Nbr)
    idx = idx + (jnp.arange(B, dtype=jnp.int32) * At)[:, None]
    idx = idx.reshape(NW, per_w // _C, _C)

    out = _build_gather(R, D, NC, per_w)(table, idx)
    return out.reshape(B, At, Nbr, Nbr, F)
